# Initial kernel scaffold; baseline (speedup 1.0000x reference)
import jax
import jax.numpy as jnp
from jax.experimental import pallas as pl


def kernel(bbox, label, rois):
    raise NotImplementedError("write your pallas kernel here")


# speedup vs baseline: 9.1634x; 9.1634x over previous
import functools

import jax
import jax.numpy as jnp
import numpy as np
from jax.experimental import pallas as pl
from jax.experimental.pallas import tpu as pltpu

N0 = 5064
NR, NL = 40, 128
NP = NR * NL
NGT = 64


def _np_threefry(k0, k1, x0, x1):
    u32 = np.uint32
    rotations = ((13, 15, 26, 6), (17, 29, 16, 24))
    with np.errstate(over="ignore"):
        ks = (k0, k1, u32(k0 ^ k1 ^ u32(0x1BD11BDA)))
        x0 = (x0 + ks[0]).astype(u32)
        x1 = (x1 + ks[1]).astype(u32)
        for i in range(5):
            for r in rotations[i % 2]:
                x0 = (x0 + x1).astype(u32)
                x1 = ((x1 << u32(r)) | (x1 >> u32(32 - r))).astype(u32)
                x1 = (x0 ^ x1).astype(u32)
            x0 = (x0 + ks[(i + 1) % 3]).astype(u32)
            x1 = (x1 + ks[(i + 2) % 3] + u32(i + 1)).astype(u32)
    return x0, x1


def _np_split(kd):
    c1 = np.zeros(2, np.uint32)
    c2 = np.arange(2, dtype=np.uint32)
    b1, b2 = _np_threefry(kd[0], kd[1], c1, c2)
    return np.array([b1[0], b2[0]], np.uint32), np.array([b1[1], b2[1]], np.uint32)


def _np_bits(kd, n):
    j = np.arange(n, dtype=np.uint32)
    o0, o1 = _np_threefry(kd[0], kd[1], np.zeros(n, np.uint32), j)
    return o0 ^ o1


def _const_streams():
    out = []
    for seed in (1, 2):
        key = np.array([0, seed], np.uint32)
        key1, s1 = _np_split(key)
        _, s2 = _np_split(key1)
        for s in (s1, s2):
            b = _np_bits(s, N0)
            o = np.argsort(b, kind="stable").astype(np.int32)
            o = np.concatenate([o, np.full((NP - N0,), 100000, np.int32)])
            out.append(o.reshape(NR, NL))
    return out


_ORD1P, _ORD2P, _ORD1N, _ORD2N = _const_streams()


def _cumsum2d(x):
    for s in (1, 2, 4, 8, 16, 32, 64):
        x = x + jnp.concatenate(
            [jnp.zeros((NR, s), x.dtype), x[:, :NL - s]], axis=1)
    rt = x[:, NL - 1:NL]
    rc = rt
    for s in (1, 2, 4, 8, 16, 32):
        rc = rc + jnp.concatenate(
            [jnp.zeros((s, 1), x.dtype), rc[:NR - s, :]], axis=0)
    return x + (rc - rt)


def _bitsel(ordv, m, c, targets):
    t3 = targets[:, None, None]
    hit = m[None, :, :] & (c[None, :, :] == t3 + 1)
    return jnp.sum(jnp.where(hit, ordv[None, :, :], 0), axis=(1, 2))


def _branch(mask, vm, idx2d, n, ord1, ord2, rvec):
    nm = (~mask) & vm
    cP = _cumsum2d(mask.astype(jnp.int32))
    cN = _cumsum2d(nm.astype(jnp.int32))
    p = jnp.where(mask, cP - 1, n + cN - 1)
    p = jnp.where(vm, p, 6000 + idx2d)
    m1 = ord1 < n
    c1 = _cumsum2d(m1.astype(jnp.int32))
    m2 = ord2 < n
    c2 = _cumsum2d(m2.astype(jnp.int32))
    pA = jnp.where(rvec < n, _bitsel(ord1, m1, c1, rvec), rvec)
    qt = jnp.where(rvec < n, _bitsel(ord2, m2, c2, rvec), rvec)
    pB = jnp.where(qt < n, _bitsel(ord1, m1, c1, qt), qt)
    pstar = jnp.where(n > 1625, pB, pA)
    hit = p[None, :, :] == pstar[:, None, None]
    return jnp.sum(jnp.where(hit, idx2d[None, :, :], 0), axis=(1, 2))


def _tb_kernel(rois_ref, gt_ref, lbl_ref, o1p_ref, o2p_ref, o1n_ref, o2n_ref,
               cls_ref, loc_ref, sroi_ref):
    x1 = rois_ref[0]
    y1 = rois_ref[1]
    x2 = rois_ref[2]
    y2 = rois_ref[3]
    gx1 = gt_ref[:, 0].reshape(NGT, 1, 1)
    gy1 = gt_ref[:, 1].reshape(NGT, 1, 1)
    gx2 = gt_ref[:, 2].reshape(NGT, 1, 1)
    gy2 = gt_ref[:, 3].reshape(NGT, 1, 1)

    iw = jnp.clip(jnp.minimum(gx2, x2[None]) - jnp.maximum(gx1, x1[None]), 0.0, None)
    ih = jnp.clip(jnp.minimum(gy2, y2[None]) - jnp.maximum(gy1, y1[None]), 0.0, None)
    inter = iw * ih
    area_r = (x2 - x1) * (y2 - y1)
    area_g = (gx2 - gx1) * (gy2 - gy1)
    union = area_r[None] + area_g - inter
    iou = inter / union

    maxv = jnp.max(iou, axis=0)
    g_iota = jax.lax.broadcasted_iota(jnp.int32, (NGT, NR, NL), 0)
    am = jnp.min(jnp.where(iou == maxv[None], g_iota, NGT), axis=0)

    idx2d = jax.lax.broadcasted_iota(jnp.int32, (NR, NL), 0) * NL + \
        jax.lax.broadcasted_iota(jnp.int32, (NR, NL), 1)
    vm = idx2d < N0
    pos_mask = (maxv >= 0.5) & vm
    neg_mask = (maxv < 0.5) & (maxv >= 0.0) & vm

    n_pos = jnp.sum(pos_mask.astype(jnp.int32))
    n_neg = jnp.sum(neg_mask.astype(jnp.int32))
    n_pos_t = jnp.minimum(jnp.sum(((maxv > 0.5) & vm).astype(jnp.int32)), 32)

    rvec = jax.lax.iota(jnp.int32, 128)
    pos_roi = _branch(pos_mask, vm, idx2d, n_pos, o1p_ref[...], o2p_ref[...], rvec)
    neg_roi = _branch(neg_mask, vm, idx2d, n_neg, o1n_ref[...], o2n_ref[...], rvec)
    keep = jnp.concatenate([pos_roi[:32], neg_roi[:96]])

    G = keep[:, None, None] == idx2d[None, :, :]

    def g2(v):
        return jnp.sum(jnp.where(G, v[None, :, :], 0.0), axis=(1, 2))

    am_k = jnp.sum(jnp.where(G, am[None, :, :], 0), axis=(1, 2))
    kx1, ky1, kx2, ky2 = g2(x1), g2(y1), g2(x2), g2(y2)

    g64 = jax.lax.iota(jnp.int32, NGT)
    GM = am_k[:, None] == g64[None, :]
    lbl = lbl_ref[0, :]

    def gg(v):
        return jnp.sum(jnp.where(GM, v[None, :], 0.0), axis=1)

    cls = jnp.sum(jnp.where(GM, lbl[None, :], 0), axis=1) + 1
    cls = jnp.where(rvec >= n_pos_t, 0, cls)

    bx1, by1 = gg(gt_ref[:, 0]), gg(gt_ref[:, 1])
    bx2, by2 = gg(gt_ref[:, 2]), gg(gt_ref[:, 3])

    pcx, pcy = (kx1 + kx2) * 0.5, (ky1 + ky2) * 0.5
    pw, ph = kx2 - kx1, ky2 - ky1
    gcx, gcy = (bx1 + bx2) * 0.5, (by1 + by2) * 0.5
    gw, gh = bx2 - bx1, by2 - by1
    tx = (gcx - pcx) / pw
    ty = (gcy - pcy) / ph
    tw = jnp.log(gw / pw)
    th = jnp.log(gh / ph)

    z = jnp.zeros((1, 128), jnp.float32)
    cls_ref[...] = jnp.concatenate(
        [cls.reshape(1, 128).astype(jnp.int32)] +
        [jnp.zeros((7, 128), jnp.int32)], axis=0)
    loc_ref[...] = jnp.concatenate(
        [tx.reshape(1, 128), ty.reshape(1, 128),
         tw.reshape(1, 128), th.reshape(1, 128), z, z, z, z], axis=0)
    sroi_ref[...] = jnp.concatenate(
        [kx1.reshape(1, 128), ky1.reshape(1, 128),
         kx2.reshape(1, 128), ky2.reshape(1, 128), z, z, z, z], axis=0)


@functools.partial(jax.jit, static_argnames=("interpret",))
def _run(bbox, label, rois, interpret=False):
    bbox = bbox[0]
    label = label[0]
    rois_all = jnp.concatenate([rois, bbox], axis=0)
    rois_pad = jnp.concatenate(
        [rois_all, jnp.zeros((NP - N0, 4), jnp.float32)], axis=0)
    rois_pl = rois_pad.T.reshape(4, NR, NL)
    gt = jnp.concatenate([bbox, jnp.zeros((NGT, 4), jnp.float32)], axis=1)[:, :8]
    lbl = jnp.zeros((8, NGT), jnp.int32).at[0].set(label.astype(jnp.int32))

    ords = [jnp.asarray(o) for o in (_ORD1P, _ORD2P, _ORD1N, _ORD2N)]

    cls8, loc8, sroi8 = pl.pallas_call(
        _tb_kernel,
        out_shape=[
            jax.ShapeDtypeStruct((8, 128), jnp.int32),
            jax.ShapeDtypeStruct((8, 128), jnp.float32),
            jax.ShapeDtypeStruct((8, 128), jnp.float32),
        ],
        interpret=interpret,
    )(rois_pl, gt, lbl, *ords)

    cls = cls8[0]
    loc = loc8[:4].T
    sroi = sroi8[:4].T
    return cls, loc, sroi


def kernel(bbox, label, rois):
    return _run(bbox, label, rois)
